# Initial kernel scaffold; baseline (speedup 1.0000x reference)
import jax
import jax.numpy as jnp
from jax.experimental import pallas as pl


def kernel(x, edge_index, edge_attr, u, batch, W0, b0, W1, b1, W2, b2, g0, be0, g1, be1, g2, be2):
    raise NotImplementedError("write your pallas kernel here")


# speedup vs baseline: 4.5047x; 4.5047x over previous
import functools

import jax
import jax.numpy as jnp
from jax import lax
from jax.experimental import pallas as pl
from jax.experimental.pallas import tpu as pltpu
from jax.experimental.pallas import tpu_sc as plsc

N = 10000
NPAD = 10240
E = 320000
D = 128
G = 64
EPS = 1e-5

NC = 2
NS = 16
NW = NC * NS
EPT = E // NW
SUB = 80
NSTEP = EPT // SUB
HR = NPAD // D
RPT = NPAD // NS
ZR = 64


def _sc_body(edge_hbm, src_hbm, z128_hbm, zflat_hbm,
             sums_out, cnt_out,
             acc, ebuf, ibuf, hist, zb128):
    c = lax.axis_index("c")
    s = lax.axis_index("s")
    wid = c * NS + s

    pltpu.sync_copy(z128_hbm, zb128)
    pltpu.sync_copy(zflat_hbm, hist)
    row0 = s * RPT

    def _zero(i, _):
        pltpu.sync_copy(zb128, acc.at[pl.ds(row0 + i * ZR, ZR), :])
        return 0

    lax.fori_loop(0, RPT // ZR, _zero, 0)
    plsc.subcore_barrier()

    ebase = wid * EPT
    ones16 = jnp.full((16,), 1.0, jnp.float32)

    def _step(i, _):
        e0 = ebase + i * SUB
        pltpu.sync_copy(src_hbm.at[pl.ds(e0, SUB)], ibuf)
        pltpu.sync_copy(edge_hbm.at[pl.ds(e0, SUB), :], ebuf)
        pltpu.sync_copy(ebuf, acc.at[ibuf], add=True)
        for j in range(SUB // 16):
            iv = ibuf[pl.ds(j * 16, 16)]
            plsc.addupdate_scatter(hist, [iv], ones16)
        return 0

    lax.fori_loop(0, NSTEP, _step, 0)
    plsc.subcore_barrier()

    def _out(i, _):
        r = row0 + i * ZR
        pltpu.sync_copy(acc.at[pl.ds(r, ZR), :], zb128)
        pltpu.sync_copy(zb128, sums_out.at[c, pl.ds(r, ZR), :])
        return 0

    lax.fori_loop(0, RPT // ZR, _out, 0)
    pltpu.sync_copy(hist, cnt_out.at[wid])


_sc_segsum = functools.partial(
    pl.kernel,
    out_type=(
        jax.ShapeDtypeStruct((NC, NPAD, D), jnp.float32),
        jax.ShapeDtypeStruct((NW, NPAD), jnp.float32),
    ),
    mesh=plsc.VectorSubcoreMesh(core_axis_name="c", subcore_axis_name="s"),
    compiler_params=pltpu.CompilerParams(needs_layout_passes=False),
    scratch_types=(
        pltpu.VMEM_SHARED((NPAD, D), jnp.float32),
        pltpu.VMEM((SUB, D), jnp.float32),
        pltpu.VMEM((SUB,), jnp.int32),
        pltpu.VMEM((NPAD,), jnp.float32),
        pltpu.VMEM((ZR, D), jnp.float32),
    ),
)(_sc_body)


def _tc_body(x_ref, s0_ref, s1_ref, ch_ref, b_ref, u_ref,
             W0_ref, W1_ref, W2_ref, b0_ref, b1_ref, b2_ref,
             s0s_ref, s1s_ref, s2s_ref, be0_ref, be1_ref, be2_ref,
             out_ref):
    cnt = jnp.sum(ch_ref[...], axis=0).reshape(BLK, 1)
    v = (s0_ref[...] + s1_ref[...]) / jnp.maximum(cnt, 1.0)
    bidx = b_ref[0, 0, :]
    onehot = (bidx[:, None]
              == lax.broadcasted_iota(jnp.int32, (1, G), 1)).astype(jnp.float32)
    uc = jnp.dot(u_ref[...], W0_ref[2 * D:3 * D, :],
                 preferred_element_type=jnp.float32)
    h = (jnp.dot(x_ref[...], W0_ref[0:D, :], preferred_element_type=jnp.float32)
         + jnp.dot(v, W0_ref[D:2 * D, :], preferred_element_type=jnp.float32)
         + jnp.dot(onehot, uc, preferred_element_type=jnp.float32)
         + b0_ref[...])
    h = jnp.maximum(h, 0.0) * s0s_ref[...] + be0_ref[...]
    h = jnp.dot(h, W1_ref[...], preferred_element_type=jnp.float32) + b1_ref[...]
    h = jnp.maximum(h, 0.0) * s1s_ref[...] + be1_ref[...]
    h = jnp.dot(h, W2_ref[...], preferred_element_type=jnp.float32) + b2_ref[...]
    h = jnp.maximum(h, 0.0) * s2s_ref[...] + be2_ref[...]
    out_ref[...] = h


BLK = 1024
NB = NPAD // BLK


def _tc_mlp(x, s0, s1, cnth, batch3, u, W0, W1, W2,
            b0, b1, b2, s0s, s1s, s2s, be0, be1, be2):
    row = lambda i: (i, 0)
    full = lambda i: (0, 0)
    vec3 = pl.BlockSpec((1, 1, BLK), lambda i: (i, 0, 0))
    return pl.pallas_call(
        _tc_body,
        grid=(NB,),
        in_specs=[
            pl.BlockSpec((BLK, D), row),
            pl.BlockSpec((BLK, D), row),
            pl.BlockSpec((BLK, D), row),
            pl.BlockSpec((NW, BLK), lambda i: (0, i)),
            vec3,
            pl.BlockSpec((G, D), full),
            pl.BlockSpec((3 * D, D), full),
            pl.BlockSpec((D, D), full),
            pl.BlockSpec((D, D), full),
        ] + [pl.BlockSpec((1, D), full)] * 9,
        out_specs=pl.BlockSpec((BLK, D), row),
        out_shape=jax.ShapeDtypeStruct((NPAD, D), jnp.float32),
    )(x, s0, s1, cnth, batch3, u, W0, W1, W2,
      b0, b1, b2, s0s, s1s, s2s, be0, be1, be2)


def kernel(x, edge_index, edge_attr, u, batch, W0, b0, W1, b1, W2, b2,
           g0, be0, g1, be1, g2, be2):
    src = edge_index[0, :]
    z128 = jnp.zeros((ZR, D), jnp.float32)
    zflat = jnp.zeros((NPAD,), jnp.float32)
    sums_p, cnth = _sc_segsum(edge_attr, src, z128, zflat)

    inv_std = 1.0 / jnp.sqrt(1.0 + EPS)
    r = lambda a: a.reshape(1, D)
    xp = jnp.zeros((NPAD, D), jnp.float32).at[:N].set(x)
    bp = jnp.zeros((NPAD,), jnp.int32).at[:N].set(batch)
    out = _tc_mlp(
        xp, sums_p[0], sums_p[1], cnth,
        bp.reshape(NB, 1, BLK), u, W0, W1, W2,
        r(b0), r(b1), r(b2),
        r(g0 * inv_std), r(g1 * inv_std), r(g2 * inv_std),
        r(be0), r(be1), r(be2))
    return out[:N]


# speedup vs baseline: 6.6948x; 1.4862x over previous
import functools

import jax
import jax.numpy as jnp
from jax import lax
from jax.experimental import pallas as pl
from jax.experimental.pallas import tpu as pltpu
from jax.experimental.pallas import tpu_sc as plsc

N = 10000
NPAD = 10240
E = 320000
D = 128
G = 64
EPS = 1e-5

NC = 2
NS = 16
NW = NC * NS
EPT = E // NW
SUB = 80
NSTEP = EPT // SUB
HR = NPAD // D
RPT = NPAD // NS
ZR = 64


def _sc_body(edge_hbm, src3_hbm, z80_hbm, zflat_hbm,
             sums_out, cnt_out,
             acc, ebuf, ibuf, hist, esem, ssem):
    c = lax.axis_index("c")
    s = lax.axis_index("s")
    wid = c * NS + s

    pltpu.sync_copy(src3_hbm.at[wid], ibuf)
    pltpu.sync_copy(z80_hbm, ebuf.at[0])
    pltpu.sync_copy(zflat_hbm, hist)
    row0 = s * RPT

    def _zero(i, _):
        pltpu.sync_copy(ebuf.at[0], acc.at[pl.ds(row0 + i * SUB, SUB), :])
        return 0

    lax.fori_loop(0, RPT // SUB, _zero, 0)
    plsc.subcore_barrier()

    ebase = wid * EPT
    ones16 = jnp.full((16,), 1.0, jnp.float32)

    def _in_copy(i, b):
        return pltpu.make_async_copy(
            edge_hbm.at[pl.ds(ebase + i * SUB, SUB), :], ebuf.at[b], esem)

    def _scat_copy(i, b):
        return pltpu.make_async_copy(ebuf.at[b], acc.at[ibuf.at[i]], ssem)

    pltpu.async_copy(edge_hbm.at[pl.ds(ebase, SUB), :], ebuf.at[0], esem)

    def _step(i, _):
        b = jnp.bitwise_and(i, 1)
        _in_copy(i, b).wait()

        @pl.when(i >= 1)
        def _():
            _scat_copy(i - 1, 1 - b).wait()

        @pl.when(i + 1 < NSTEP)
        def _():
            pltpu.async_copy(edge_hbm.at[pl.ds(ebase + (i + 1) * SUB, SUB), :],
                             ebuf.at[1 - b], esem)

        pltpu.async_copy(ebuf.at[b], acc.at[ibuf.at[i]], ssem, add=True)
        for j in range(SUB // 16):
            iv = ibuf[i, pl.ds(j * 16, 16)]
            plsc.addupdate_scatter(hist, [iv], ones16)
        return 0

    lax.fori_loop(0, NSTEP, _step, 0)
    _scat_copy(NSTEP - 1, (NSTEP - 1) & 1).wait()
    plsc.subcore_barrier()

    def _out(i, _):
        r = row0 + i * SUB
        pltpu.sync_copy(acc.at[pl.ds(r, SUB), :], ebuf.at[0])
        pltpu.sync_copy(ebuf.at[0], sums_out.at[c, pl.ds(r, SUB), :])
        return 0

    lax.fori_loop(0, RPT // SUB, _out, 0)
    pltpu.sync_copy(hist, cnt_out.at[wid])


_sc_segsum = functools.partial(
    pl.kernel,
    out_type=(
        jax.ShapeDtypeStruct((NC, NPAD, D), jnp.float32),
        jax.ShapeDtypeStruct((NW, NPAD), jnp.float32),
    ),
    mesh=plsc.VectorSubcoreMesh(core_axis_name="c", subcore_axis_name="s"),
    compiler_params=pltpu.CompilerParams(needs_layout_passes=False),
    scratch_types=(
        pltpu.VMEM_SHARED((NPAD, D), jnp.float32),
        pltpu.VMEM((2, SUB, D), jnp.float32),
        pltpu.VMEM((NSTEP, SUB), jnp.int32),
        pltpu.VMEM((NPAD,), jnp.float32),
        pltpu.SemaphoreType.DMA,
        pltpu.SemaphoreType.DMA,
    ),
)(_sc_body)


def _tc_body(x_ref, s0_ref, s1_ref, ch_ref, b_ref, u_ref,
             W0_ref, W1_ref, W2_ref, b0_ref, b1_ref, b2_ref,
             s0s_ref, s1s_ref, s2s_ref, be0_ref, be1_ref, be2_ref,
             out_ref):
    cnt = jnp.sum(ch_ref[...], axis=0).reshape(BLK, 1)
    v = (s0_ref[...] + s1_ref[...]) / jnp.maximum(cnt, 1.0)
    bidx = b_ref[0, 0, :]
    onehot = (bidx[:, None]
              == lax.broadcasted_iota(jnp.int32, (1, G), 1)).astype(jnp.float32)
    uc = jnp.dot(u_ref[...], W0_ref[2 * D:3 * D, :],
                 preferred_element_type=jnp.float32)
    h = (jnp.dot(x_ref[...], W0_ref[0:D, :], preferred_element_type=jnp.float32)
         + jnp.dot(v, W0_ref[D:2 * D, :], preferred_element_type=jnp.float32)
         + jnp.dot(onehot, uc, preferred_element_type=jnp.float32)
         + b0_ref[...])
    h = jnp.maximum(h, 0.0) * s0s_ref[...] + be0_ref[...]
    h = jnp.dot(h, W1_ref[...], preferred_element_type=jnp.float32) + b1_ref[...]
    h = jnp.maximum(h, 0.0) * s1s_ref[...] + be1_ref[...]
    h = jnp.dot(h, W2_ref[...], preferred_element_type=jnp.float32) + b2_ref[...]
    h = jnp.maximum(h, 0.0) * s2s_ref[...] + be2_ref[...]
    out_ref[...] = h


BLK = 1024
NB = NPAD // BLK


def _tc_mlp(x, s0, s1, cnth, batch3, u, W0, W1, W2,
            b0, b1, b2, s0s, s1s, s2s, be0, be1, be2):
    row = lambda i: (i, 0)
    full = lambda i: (0, 0)
    vec3 = pl.BlockSpec((1, 1, BLK), lambda i: (i, 0, 0))
    return pl.pallas_call(
        _tc_body,
        grid=(NB,),
        in_specs=[
            pl.BlockSpec((BLK, D), row),
            pl.BlockSpec((BLK, D), row),
            pl.BlockSpec((BLK, D), row),
            pl.BlockSpec((NW, BLK), lambda i: (0, i)),
            vec3,
            pl.BlockSpec((G, D), full),
            pl.BlockSpec((3 * D, D), full),
            pl.BlockSpec((D, D), full),
            pl.BlockSpec((D, D), full),
        ] + [pl.BlockSpec((1, D), full)] * 9,
        out_specs=pl.BlockSpec((BLK, D), row),
        out_shape=jax.ShapeDtypeStruct((NPAD, D), jnp.float32),
    )(x, s0, s1, cnth, batch3, u, W0, W1, W2,
      b0, b1, b2, s0s, s1s, s2s, be0, be1, be2)


def kernel(x, edge_index, edge_attr, u, batch, W0, b0, W1, b1, W2, b2,
           g0, be0, g1, be1, g2, be2):
    src3 = edge_index[0, :].reshape(NW, NSTEP, SUB)
    z80 = jnp.zeros((SUB, D), jnp.float32)
    zflat = jnp.zeros((NPAD,), jnp.float32)
    sums_p, cnth = _sc_segsum(edge_attr, src3, z80, zflat)

    inv_std = 1.0 / jnp.sqrt(1.0 + EPS)
    r = lambda a: a.reshape(1, D)
    xp = jnp.zeros((NPAD, D), jnp.float32).at[:N].set(x)
    bp = jnp.zeros((NPAD,), jnp.int32).at[:N].set(batch)
    out = _tc_mlp(
        xp, sums_p[0], sums_p[1], cnth,
        bp.reshape(NB, 1, BLK), u, W0, W1, W2,
        r(b0), r(b1), r(b2),
        r(g0 * inv_std), r(g1 * inv_std), r(g2 * inv_std),
        r(be0), r(be1), r(be2))
    return out[:N]


# speedup vs baseline: 6.8980x; 1.0303x over previous
import functools

import jax
import jax.numpy as jnp
from jax import lax
from jax.experimental import pallas as pl
from jax.experimental.pallas import tpu as pltpu
from jax.experimental.pallas import tpu_sc as plsc

N = 10000
NPAD = 10240
E = 320000
D = 128
G = 64
EPS = 1e-5

NC = 2
NS = 16
NW = NC * NS
EPT = E // NW
SUB = 80
NSTEP = EPT // SUB
HR = NPAD // D
RPT = NPAD // NS
ZR = 64


def _sc_body(edge_hbm, src3_hbm, z80_hbm, zflat_hbm,
             sums_out, cnt_out,
             acc, ebuf, ibuf, hist, esem, ssem):
    c = lax.axis_index("c")
    s = lax.axis_index("s")
    wid = c * NS + s

    pltpu.sync_copy(src3_hbm.at[wid], ibuf)
    pltpu.sync_copy(z80_hbm, ebuf.at[0])
    pltpu.sync_copy(zflat_hbm, hist)
    row0 = s * RPT

    def _zero(i, _):
        pltpu.sync_copy(ebuf.at[0], acc.at[pl.ds(row0 + i * SUB, SUB), :])
        return 0

    lax.fori_loop(0, RPT // SUB, _zero, 0)
    plsc.subcore_barrier()

    ebase = wid * EPT
    ones16 = jnp.full((16,), 1.0, jnp.float32)

    def _in_copy(i, b):
        return pltpu.make_async_copy(
            edge_hbm.at[pl.ds(ebase + i * SUB, SUB), :], ebuf.at[b], esem)

    def _scat_copy(i, b):
        return pltpu.make_async_copy(ebuf.at[b], acc.at[ibuf.at[i]], ssem)

    pltpu.async_copy(edge_hbm.at[pl.ds(ebase, SUB), :], ebuf.at[0], esem)

    def _step(i, _):
        b = jnp.bitwise_and(i, 1)
        _in_copy(i, b).wait()

        @pl.when(i >= 1)
        def _():
            _scat_copy(i - 1, 1 - b).wait()

        @pl.when(i + 1 < NSTEP)
        def _():
            pltpu.async_copy(edge_hbm.at[pl.ds(ebase + (i + 1) * SUB, SUB), :],
                             ebuf.at[1 - b], esem)

        pltpu.async_copy(ebuf.at[b], acc.at[ibuf.at[i]], ssem, add=True)
        for j in range(SUB // 16):
            iv = ibuf[i, pl.ds(j * 16, 16)]
            plsc.addupdate_scatter(hist, [iv], ones16)
        return 0

    lax.fori_loop(0, NSTEP, _step, 0)
    _scat_copy(NSTEP - 1, (NSTEP - 1) & 1).wait()
    plsc.subcore_barrier()

    def _out(i, _):
        r = row0 + i * SUB
        pltpu.sync_copy(acc.at[pl.ds(r, SUB), :], ebuf.at[0])
        pltpu.sync_copy(ebuf.at[0], sums_out.at[c, pl.ds(r, SUB), :])
        return 0

    lax.fori_loop(0, RPT // SUB, _out, 0)
    pltpu.sync_copy(hist, cnt_out.at[wid])


_sc_segsum = functools.partial(
    pl.kernel,
    out_type=(
        jax.ShapeDtypeStruct((NC, NPAD, D), jnp.float32),
        jax.ShapeDtypeStruct((NW, NPAD), jnp.float32),
    ),
    mesh=plsc.VectorSubcoreMesh(core_axis_name="c", subcore_axis_name="s"),
    compiler_params=pltpu.CompilerParams(needs_layout_passes=False),
    scratch_types=(
        pltpu.VMEM_SHARED((NPAD, D), jnp.float32),
        pltpu.VMEM((2, SUB, D), jnp.float32),
        pltpu.VMEM((NSTEP, SUB), jnp.int32),
        pltpu.VMEM((NPAD,), jnp.float32),
        pltpu.SemaphoreType.DMA,
        pltpu.SemaphoreType.DMA,
    ),
)(_sc_body)


def _tc_body(x_ref, s0_ref, s1_ref, ch_ref, b_ref, u_ref,
             W0_ref, W1_ref, W2_ref, b0_ref, b1_ref, b2_ref,
             s0s_ref, s1s_ref, s2s_ref, be0_ref, be1_ref, be2_ref,
             out_ref):
    cnt = jnp.sum(ch_ref[...], axis=0).reshape(BLK, 1)
    v = (s0_ref[...] + s1_ref[...]) / jnp.maximum(cnt, 1.0)
    bidx = b_ref[0, 0, :]
    onehot = (bidx[:, None]
              == lax.broadcasted_iota(jnp.int32, (1, G), 1)).astype(jnp.float32)
    uc = jnp.dot(u_ref[...], W0_ref[2 * D:3 * D, :],
                 preferred_element_type=jnp.float32)
    h = (jnp.dot(x_ref[...], W0_ref[0:D, :], preferred_element_type=jnp.float32)
         + jnp.dot(v, W0_ref[D:2 * D, :], preferred_element_type=jnp.float32)
         + jnp.dot(onehot, uc, preferred_element_type=jnp.float32)
         + b0_ref[...])
    h = jnp.maximum(h, 0.0) * s0s_ref[...] + be0_ref[...]
    h = jnp.dot(h, W1_ref[...], preferred_element_type=jnp.float32) + b1_ref[...]
    h = jnp.maximum(h, 0.0) * s1s_ref[...] + be1_ref[...]
    h = jnp.dot(h, W2_ref[...], preferred_element_type=jnp.float32) + b2_ref[...]
    h = jnp.maximum(h, 0.0) * s2s_ref[...] + be2_ref[...]
    out_ref[...] = h


BLK = 1024
NB = -(-N // BLK)


def _tc_mlp(x, s0, s1, cnth, batch3, u, W0, W1, W2,
            b0, b1, b2, s0s, s1s, s2s, be0, be1, be2):
    row = lambda i: (i, 0)
    full = lambda i: (0, 0)
    vec3 = pl.BlockSpec((1, 1, BLK), lambda i: (i, 0, 0))
    return pl.pallas_call(
        _tc_body,
        grid=(NB,),
        in_specs=[
            pl.BlockSpec((BLK, D), row),
            pl.BlockSpec((BLK, D), row),
            pl.BlockSpec((BLK, D), row),
            pl.BlockSpec((NW, BLK), lambda i: (0, i)),
            vec3,
            pl.BlockSpec((G, D), full),
            pl.BlockSpec((3 * D, D), full),
            pl.BlockSpec((D, D), full),
            pl.BlockSpec((D, D), full),
        ] + [pl.BlockSpec((1, D), full)] * 9,
        out_specs=pl.BlockSpec((BLK, D), row),
        out_shape=jax.ShapeDtypeStruct((N, D), jnp.float32),
    )(x, s0, s1, cnth, batch3, u, W0, W1, W2,
      b0, b1, b2, s0s, s1s, s2s, be0, be1, be2)


def kernel(x, edge_index, edge_attr, u, batch, W0, b0, W1, b1, W2, b2,
           g0, be0, g1, be1, g2, be2):
    src3 = edge_index[0, :].reshape(NW, NSTEP, SUB)
    z80 = jnp.zeros((SUB, D), jnp.float32)
    zflat = jnp.zeros((NPAD,), jnp.float32)
    sums_p, cnth = _sc_segsum(edge_attr, src3, z80, zflat)

    inv_std = 1.0 / jnp.sqrt(1.0 + EPS)
    r = lambda a: a.reshape(1, D)
    bp = jnp.zeros((NB * BLK,), jnp.int32).at[:N].set(batch)
    return _tc_mlp(
        x, sums_p[0], sums_p[1], cnth,
        bp.reshape(NB, 1, BLK), u, W0, W1, W2,
        r(b0), r(b1), r(b2),
        r(g0 * inv_std), r(g1 * inv_std), r(g2 * inv_std),
        r(be0), r(be1), r(be2))


# speedup vs baseline: 8.3799x; 1.2148x over previous
import functools

import jax
import jax.numpy as jnp
from jax import lax
from jax.experimental import pallas as pl
from jax.experimental.pallas import tpu as pltpu
from jax.experimental.pallas import tpu_sc as plsc

N = 10000
NPAD = 10240
E = 320000
D = 128
G = 64
EPS = 1e-5

NC = 2
NS = 16
NW = NC * NS
EPT = E // NW
SUB = 80
NSTEP = EPT // SUB
HR = NPAD // D
RPT = NPAD // NS
ZR = 64


def _sc_body(edge_hbm, src3_hbm, z80_hbm, zflat_hbm,
             sums_out, cnt_out,
             acc, ebuf, ibuf, hist, esem, ssem):
    c = lax.axis_index("c")
    s = lax.axis_index("s")
    wid = c * NS + s

    pltpu.sync_copy(src3_hbm.at[wid], ibuf)
    pltpu.sync_copy(z80_hbm, ebuf.at[0])
    pltpu.sync_copy(zflat_hbm, hist)
    row0 = s * RPT

    def _zero(i, _):
        pltpu.sync_copy(ebuf.at[0], acc.at[pl.ds(row0 + i * SUB, SUB), :])
        return 0

    lax.fori_loop(0, RPT // SUB, _zero, 0)
    plsc.subcore_barrier()

    ebase = wid * EPT
    ones16 = jnp.full((16,), 1.0, jnp.float32)

    H = SUB // 2

    def _in_fire(i, b):
        e0 = ebase + i * SUB
        pltpu.async_copy(edge_hbm.at[pl.ds(e0, H), :],
                         ebuf.at[b, pl.ds(0, H), :], esem)
        pltpu.async_copy(edge_hbm.at[pl.ds(e0 + H, H), :],
                         ebuf.at[b, pl.ds(H, H), :], esem)

    def _in_wait(i, b):
        e0 = ebase + i * SUB
        pltpu.make_async_copy(edge_hbm.at[pl.ds(e0, H), :],
                              ebuf.at[b, pl.ds(0, H), :], esem).wait()
        pltpu.make_async_copy(edge_hbm.at[pl.ds(e0 + H, H), :],
                              ebuf.at[b, pl.ds(H, H), :], esem).wait()

    def _scat_copy(i, b):
        return pltpu.make_async_copy(ebuf.at[b], acc.at[ibuf.at[i]], ssem)

    _in_fire(0, 0)

    def _step(i, _):
        b = jnp.bitwise_and(i, 1)

        @pl.when(i >= 1)
        def _():
            _scat_copy(i - 1, 1 - b).wait()

        @pl.when(i + 1 < NSTEP)
        def _():
            _in_fire(i + 1, 1 - b)

        _in_wait(i, b)
        pltpu.async_copy(ebuf.at[b], acc.at[ibuf.at[i]], ssem, add=True)
        for j in range(SUB // 16):
            iv = ibuf[i, pl.ds(j * 16, 16)]
            plsc.addupdate_scatter(hist, [iv], ones16)
        return 0

    lax.fori_loop(0, NSTEP, _step, 0)
    _scat_copy(NSTEP - 1, (NSTEP - 1) & 1).wait()
    plsc.subcore_barrier()

    def _out(i, _):
        r = row0 + i * SUB
        pltpu.sync_copy(acc.at[pl.ds(r, SUB), :], ebuf.at[0])
        pltpu.sync_copy(ebuf.at[0], sums_out.at[c, pl.ds(r, SUB), :])
        return 0

    lax.fori_loop(0, RPT // SUB, _out, 0)
    pltpu.sync_copy(hist, cnt_out.at[wid])


_sc_segsum = functools.partial(
    pl.kernel,
    out_type=(
        jax.ShapeDtypeStruct((NC, NPAD, D), jnp.float32),
        jax.ShapeDtypeStruct((NW, NPAD), jnp.float32),
    ),
    mesh=plsc.VectorSubcoreMesh(core_axis_name="c", subcore_axis_name="s"),
    compiler_params=pltpu.CompilerParams(needs_layout_passes=False),
    scratch_types=(
        pltpu.VMEM_SHARED((NPAD, D), jnp.float32),
        pltpu.VMEM((2, SUB, D), jnp.float32),
        pltpu.VMEM((NSTEP, SUB), jnp.int32),
        pltpu.VMEM((NPAD,), jnp.float32),
        pltpu.SemaphoreType.DMA,
        pltpu.SemaphoreType.DMA,
    ),
)(_sc_body)


def _tc_body(x_ref, s0_ref, s1_ref, ch_ref, b_ref, u_ref,
             W0_ref, W1_ref, W2_ref, b0_ref, b1_ref, b2_ref,
             s0s_ref, s1s_ref, s2s_ref, be0_ref, be1_ref, be2_ref,
             out_ref):
    cnt = jnp.sum(ch_ref[...], axis=0).reshape(BLK, 1)
    v = (s0_ref[...] + s1_ref[...]) / jnp.maximum(cnt, 1.0)
    bidx = b_ref[0, 0, :]
    onehot = (bidx[:, None]
              == lax.broadcasted_iota(jnp.int32, (1, G), 1)).astype(jnp.float32)
    uc = jnp.dot(u_ref[...], W0_ref[2 * D:3 * D, :],
                 preferred_element_type=jnp.float32)
    h = (jnp.dot(x_ref[...], W0_ref[0:D, :], preferred_element_type=jnp.float32)
         + jnp.dot(v, W0_ref[D:2 * D, :], preferred_element_type=jnp.float32)
         + jnp.dot(onehot, uc, preferred_element_type=jnp.float32)
         + b0_ref[...])
    h = jnp.maximum(h, 0.0) * s0s_ref[...] + be0_ref[...]
    h = jnp.dot(h, W1_ref[...], preferred_element_type=jnp.float32) + b1_ref[...]
    h = jnp.maximum(h, 0.0) * s1s_ref[...] + be1_ref[...]
    h = jnp.dot(h, W2_ref[...], preferred_element_type=jnp.float32) + b2_ref[...]
    h = jnp.maximum(h, 0.0) * s2s_ref[...] + be2_ref[...]
    out_ref[...] = h


BLK = 1024
NB = -(-N // BLK)


def _tc_mlp(x, s0, s1, cnth, batch3, u, W0, W1, W2,
            b0, b1, b2, s0s, s1s, s2s, be0, be1, be2):
    row = lambda i: (i, 0)
    full = lambda i: (0, 0)
    vec3 = pl.BlockSpec((1, 1, BLK), lambda i: (i, 0, 0))
    return pl.pallas_call(
        _tc_body,
        grid=(NB,),
        in_specs=[
            pl.BlockSpec((BLK, D), row),
            pl.BlockSpec((BLK, D), row),
            pl.BlockSpec((BLK, D), row),
            pl.BlockSpec((NW, BLK), lambda i: (0, i)),
            vec3,
            pl.BlockSpec((G, D), full),
            pl.BlockSpec((3 * D, D), full),
            pl.BlockSpec((D, D), full),
            pl.BlockSpec((D, D), full),
        ] + [pl.BlockSpec((1, D), full)] * 9,
        out_specs=pl.BlockSpec((BLK, D), row),
        out_shape=jax.ShapeDtypeStruct((N, D), jnp.float32),
    )(x, s0, s1, cnth, batch3, u, W0, W1, W2,
      b0, b1, b2, s0s, s1s, s2s, be0, be1, be2)


def kernel(x, edge_index, edge_attr, u, batch, W0, b0, W1, b1, W2, b2,
           g0, be0, g1, be1, g2, be2):
    src3 = edge_index[0, :].reshape(NW, NSTEP, SUB)
    z80 = jnp.zeros((SUB, D), jnp.float32)
    zflat = jnp.zeros((NPAD,), jnp.float32)
    sums_p, cnth = _sc_segsum(edge_attr, src3, z80, zflat)

    inv_std = 1.0 / jnp.sqrt(1.0 + EPS)
    r = lambda a: a.reshape(1, D)
    bp = jnp.zeros((NB * BLK,), jnp.int32).at[:N].set(batch)
    return _tc_mlp(
        x, sums_p[0], sums_p[1], cnth,
        bp.reshape(NB, 1, BLK), u, W0, W1, W2,
        r(b0), r(b1), r(b2),
        r(g0 * inv_std), r(g1 * inv_std), r(g2 * inv_std),
        r(be0), r(be1), r(be2))


# speedup vs baseline: 9.4353x; 1.1259x over previous
import functools

import jax
import jax.numpy as jnp
from jax import lax
from jax.experimental import pallas as pl
from jax.experimental.pallas import tpu as pltpu
from jax.experimental.pallas import tpu_sc as plsc

N = 10000
NPAD = 10240
E = 320000
D = 128
G = 64
EPS = 1e-5

NC = 2
NS = 16
NW = NC * NS
EPT = E // NW
SUB = 80
NSTEP = EPT // SUB
IGRP = 32
NGRP = 4
HR = NPAD // D
RPT = NPAD // NS
ZR = 64


def _sc_body(edge_hbm, src3_hbm, z80_hbm, zflat_hbm,
             sums_out, cnt_out,
             acc, ebuf, ibuf, hist, esem, isem, ssem):
    c = lax.axis_index("c")
    s = lax.axis_index("s")
    wid = c * NS + s

    pltpu.sync_copy(src3_hbm.at[wid, pl.ds(0, IGRP)], ibuf.at[0])
    pltpu.sync_copy(z80_hbm, ebuf.at[0])
    pltpu.sync_copy(zflat_hbm, hist)
    row0 = s * RPT

    def _zero(i, _):
        pltpu.sync_copy(ebuf.at[0], acc.at[pl.ds(row0 + i * SUB, SUB), :])
        return 0

    lax.fori_loop(0, RPT // SUB, _zero, 0)
    plsc.subcore_barrier()

    ebase = wid * EPT
    ones16 = jnp.full((16,), 1.0, jnp.float32)

    H = SUB // 2

    def _in_fire(i, b):
        e0 = ebase + i * SUB
        pltpu.async_copy(edge_hbm.at[pl.ds(e0, H), :],
                         ebuf.at[b, pl.ds(0, H), :], esem.at[b])
        pltpu.async_copy(edge_hbm.at[pl.ds(e0 + H, H), :],
                         ebuf.at[b, pl.ds(H, H), :], esem.at[b])

    def _in_wait(i, b):
        e0 = ebase + i * SUB
        pltpu.make_async_copy(edge_hbm.at[pl.ds(e0, H), :],
                              ebuf.at[b, pl.ds(0, H), :], esem.at[b]).wait()
        pltpu.make_async_copy(edge_hbm.at[pl.ds(e0 + H, H), :],
                              ebuf.at[b, pl.ds(H, H), :], esem.at[b]).wait()

    def _irow(i):
        return ibuf.at[jnp.bitwise_and(i // IGRP, 1), jnp.remainder(i, IGRP)]

    def _scat_wait(i, b):
        pltpu.make_async_copy(ebuf.at[b], acc.at[_irow(i)], ssem).wait()

    _in_fire(0, 0)
    _in_fire(1, 1)

    def _step(i, _):
        b = jnp.remainder(i, 3)

        @pl.when(i >= 1)
        def _():
            _scat_wait(i - 1, jnp.remainder(i + 2, 3))

        @pl.when(jnp.remainder(i, IGRP) == 0)
        def _():
            g = i // IGRP

            @pl.when(g >= 1)
            def _():
                pltpu.make_async_copy(
                    src3_hbm.at[wid, pl.ds(g * IGRP, IGRP)],
                    ibuf.at[jnp.bitwise_and(g, 1)], isem).wait()

            @pl.when((g + 1) * IGRP < NGRP * IGRP)
            def _():
                pltpu.async_copy(
                    src3_hbm.at[wid, pl.ds((g + 1) * IGRP, IGRP)],
                    ibuf.at[jnp.bitwise_and(g + 1, 1)], isem)

        @pl.when(i + 2 < NSTEP)
        def _():
            _in_fire(i + 2, jnp.remainder(i + 2, 3))

        _in_wait(i, b)
        pltpu.async_copy(ebuf.at[b], acc.at[_irow(i)], ssem, add=True)
        for j in range(SUB // 16):
            iv = ibuf[jnp.bitwise_and(i // IGRP, 1),
                      jnp.remainder(i, IGRP), pl.ds(j * 16, 16)]
            plsc.addupdate_scatter(hist, [iv], ones16)
        return 0

    lax.fori_loop(0, NSTEP, _step, 0)
    _scat_wait(NSTEP - 1, (NSTEP - 1) % 3)
    plsc.subcore_barrier()

    def _out(i, _):
        r = row0 + i * SUB
        pltpu.sync_copy(acc.at[pl.ds(r, SUB), :], ebuf.at[0])
        pltpu.sync_copy(ebuf.at[0], sums_out.at[c, pl.ds(r, SUB), :])
        return 0

    lax.fori_loop(0, RPT // SUB, _out, 0)
    pltpu.sync_copy(hist, cnt_out.at[wid])


_sc_segsum = functools.partial(
    pl.kernel,
    out_type=(
        jax.ShapeDtypeStruct((NC, NPAD, D), jnp.float32),
        jax.ShapeDtypeStruct((NW, HR * D - D), jnp.float32),
    ),
    mesh=plsc.VectorSubcoreMesh(core_axis_name="c", subcore_axis_name="s"),
    compiler_params=pltpu.CompilerParams(needs_layout_passes=False),
    scratch_types=(
        pltpu.VMEM_SHARED((NPAD, D), jnp.float32),
        pltpu.VMEM((3, SUB, D), jnp.float32),
        pltpu.VMEM((2, IGRP, SUB), jnp.int32),
        pltpu.VMEM((HR * D - D,), jnp.float32),
        pltpu.SemaphoreType.DMA((3,)),
        pltpu.SemaphoreType.DMA,
        pltpu.SemaphoreType.DMA,
    ),
)(_sc_body)


def _tc_body(x_ref, s0_ref, s1_ref, ch_ref, b_ref, u_ref,
             W0_ref, W1_ref, W2_ref, b0_ref, b1_ref, b2_ref,
             s0s_ref, s1s_ref, s2s_ref, be0_ref, be1_ref, be2_ref,
             out_ref):
    cnt = jnp.sum(ch_ref[...], axis=0).reshape(BLK, 1)
    v = (s0_ref[...] + s1_ref[...]) / jnp.maximum(cnt, 1.0)
    bidx = b_ref[0, 0, :]
    onehot = (bidx[:, None]
              == lax.broadcasted_iota(jnp.int32, (1, G), 1)).astype(jnp.float32)
    uc = jnp.dot(u_ref[...], W0_ref[2 * D:3 * D, :],
                 preferred_element_type=jnp.float32)
    h = (jnp.dot(x_ref[...], W0_ref[0:D, :], preferred_element_type=jnp.float32)
         + jnp.dot(v, W0_ref[D:2 * D, :], preferred_element_type=jnp.float32)
         + jnp.dot(onehot, uc, preferred_element_type=jnp.float32)
         + b0_ref[...])
    h = jnp.maximum(h, 0.0) * s0s_ref[...] + be0_ref[...]
    h = jnp.dot(h, W1_ref[...], preferred_element_type=jnp.float32) + b1_ref[...]
    h = jnp.maximum(h, 0.0) * s1s_ref[...] + be1_ref[...]
    h = jnp.dot(h, W2_ref[...], preferred_element_type=jnp.float32) + b2_ref[...]
    h = jnp.maximum(h, 0.0) * s2s_ref[...] + be2_ref[...]
    out_ref[...] = h


BLK = 1024
NB = -(-N // BLK)


def _tc_mlp(x, s0, s1, cnth, batch3, u, W0, W1, W2,
            b0, b1, b2, s0s, s1s, s2s, be0, be1, be2):
    row = lambda i: (i, 0)
    full = lambda i: (0, 0)
    vec3 = pl.BlockSpec((1, 1, BLK), lambda i: (i, 0, 0))
    return pl.pallas_call(
        _tc_body,
        grid=(NB,),
        in_specs=[
            pl.BlockSpec((BLK, D), row),
            pl.BlockSpec((BLK, D), row),
            pl.BlockSpec((BLK, D), row),
            pl.BlockSpec((NW, BLK), lambda i: (0, i)),
            vec3,
            pl.BlockSpec((G, D), full),
            pl.BlockSpec((3 * D, D), full),
            pl.BlockSpec((D, D), full),
            pl.BlockSpec((D, D), full),
        ] + [pl.BlockSpec((1, D), full)] * 9,
        out_specs=pl.BlockSpec((BLK, D), row),
        out_shape=jax.ShapeDtypeStruct((N, D), jnp.float32),
    )(x, s0, s1, cnth, batch3, u, W0, W1, W2,
      b0, b1, b2, s0s, s1s, s2s, be0, be1, be2)


def kernel(x, edge_index, edge_attr, u, batch, W0, b0, W1, b1, W2, b2,
           g0, be0, g1, be1, g2, be2):
    src3 = edge_index[0, :].reshape(NW, NSTEP, SUB)
    src3 = jnp.concatenate(
        [src3, jnp.zeros((NW, NGRP * IGRP - NSTEP, SUB), jnp.int32)], axis=1)
    z80 = jnp.zeros((SUB, D), jnp.float32)
    zflat = jnp.zeros((HR * D - D,), jnp.float32)
    sums_p, cnth = _sc_segsum(edge_attr, src3, z80, zflat)

    inv_std = 1.0 / jnp.sqrt(1.0 + EPS)
    r = lambda a: a.reshape(1, D)
    bp = jnp.zeros((NB * BLK,), jnp.int32).at[:N].set(batch)
    return _tc_mlp(
        x, sums_p[0], sums_p[1], cnth,
        bp.reshape(NB, 1, BLK), u, W0, W1, W2,
        r(b0), r(b1), r(b2),
        r(g0 * inv_std), r(g1 * inv_std), r(g2 * inv_std),
        r(be0), r(be1), r(be2))


# speedup vs baseline: 9.5489x; 1.0120x over previous
import functools

import jax
import jax.numpy as jnp
from jax import lax
from jax.experimental import pallas as pl
from jax.experimental.pallas import tpu as pltpu
from jax.experimental.pallas import tpu_sc as plsc

N = 10000
NPAD = 10240
E = 320000
D = 128
G = 64
EPS = 1e-5

NC = 2
NS = 16
NW = NC * NS
EPT = E // NW
SUB = 80
NSTEP = EPT // SUB
IGRP = 32
NGRP = 4
HR = NPAD // D
RPT = NPAD // NS
ZR = 64


def _sc_body(edge_hbm, src3_hbm, z80_hbm, zflat_hbm,
             sums_out, cnt_out,
             acc, ebuf, ibuf, hist, esem, isem, ssem):
    c = lax.axis_index("c")
    s = lax.axis_index("s")
    wid = c * NS + s

    pltpu.sync_copy(src3_hbm.at[wid, pl.ds(0, IGRP)], ibuf.at[0])
    pltpu.sync_copy(z80_hbm, ebuf.at[0])
    pltpu.sync_copy(zflat_hbm, hist)
    row0 = s * RPT

    def _zero(i, _):
        pltpu.async_copy(ebuf.at[0], acc.at[pl.ds(row0 + i * SUB, SUB), :],
                         isem)
        return 0

    lax.fori_loop(0, RPT // SUB, _zero, 0)

    def _zwait(i, _):
        pltpu.make_async_copy(ebuf.at[0],
                              acc.at[pl.ds(row0 + i * SUB, SUB), :],
                              isem).wait()
        return 0

    lax.fori_loop(0, RPT // SUB, _zwait, 0)
    plsc.subcore_barrier()

    ebase = wid * EPT
    ones16 = jnp.full((16,), 1.0, jnp.float32)

    H = SUB // 2

    def _in_fire(i, b):
        e0 = ebase + i * SUB
        pltpu.async_copy(edge_hbm.at[pl.ds(e0, H), :],
                         ebuf.at[b, pl.ds(0, H), :], esem.at[b])
        pltpu.async_copy(edge_hbm.at[pl.ds(e0 + H, H), :],
                         ebuf.at[b, pl.ds(H, H), :], esem.at[b])

    def _in_wait(i, b):
        e0 = ebase + i * SUB
        pltpu.make_async_copy(edge_hbm.at[pl.ds(e0, H), :],
                              ebuf.at[b, pl.ds(0, H), :], esem.at[b]).wait()
        pltpu.make_async_copy(edge_hbm.at[pl.ds(e0 + H, H), :],
                              ebuf.at[b, pl.ds(H, H), :], esem.at[b]).wait()

    def _irow(i):
        return ibuf.at[jnp.bitwise_and(i // IGRP, 1), jnp.remainder(i, IGRP)]

    def _scat_wait(i, b):
        pltpu.make_async_copy(ebuf.at[b], acc.at[_irow(i)], ssem).wait()

    _in_fire(0, 0)
    _in_fire(1, 1)

    def _step(i, _):
        b = jnp.remainder(i, 3)

        @pl.when(i >= 1)
        def _():
            _scat_wait(i - 1, jnp.remainder(i + 2, 3))

        @pl.when(jnp.remainder(i, IGRP) == 0)
        def _():
            g = i // IGRP

            @pl.when(g >= 1)
            def _():
                pltpu.make_async_copy(
                    src3_hbm.at[wid, pl.ds(g * IGRP, IGRP)],
                    ibuf.at[jnp.bitwise_and(g, 1)], isem).wait()

            @pl.when((g + 1) * IGRP < NGRP * IGRP)
            def _():
                pltpu.async_copy(
                    src3_hbm.at[wid, pl.ds((g + 1) * IGRP, IGRP)],
                    ibuf.at[jnp.bitwise_and(g + 1, 1)], isem)

        @pl.when(i + 2 < NSTEP)
        def _():
            _in_fire(i + 2, jnp.remainder(i + 2, 3))

        _in_wait(i, b)
        pltpu.async_copy(ebuf.at[b], acc.at[_irow(i)], ssem, add=True)
        for j in range(SUB // 16):
            iv = ibuf[jnp.bitwise_and(i // IGRP, 1),
                      jnp.remainder(i, IGRP), pl.ds(j * 16, 16)]
            plsc.addupdate_scatter(hist, [iv], ones16)
        return 0

    lax.fori_loop(0, NSTEP, _step, 0)
    _scat_wait(NSTEP - 1, (NSTEP - 1) % 3)
    plsc.subcore_barrier()

    pltpu.async_copy(hist, cnt_out.at[wid], isem)

    def _out(i, _):
        r = row0 + i * SUB
        pltpu.async_copy(acc.at[pl.ds(r, SUB), :],
                         sums_out.at[c, pl.ds(r, SUB), :], esem.at[0])
        return 0

    lax.fori_loop(0, RPT // SUB, _out, 0)

    def _owait(i, _):
        r = row0 + i * SUB
        pltpu.make_async_copy(acc.at[pl.ds(r, SUB), :],
                              sums_out.at[c, pl.ds(r, SUB), :],
                              esem.at[0]).wait()
        return 0

    lax.fori_loop(0, RPT // SUB, _owait, 0)
    pltpu.make_async_copy(hist, cnt_out.at[wid], isem).wait()


_sc_segsum = functools.partial(
    pl.kernel,
    out_type=(
        jax.ShapeDtypeStruct((NC, NPAD, D), jnp.float32),
        jax.ShapeDtypeStruct((NW, HR * D - D), jnp.float32),
    ),
    mesh=plsc.VectorSubcoreMesh(core_axis_name="c", subcore_axis_name="s"),
    compiler_params=pltpu.CompilerParams(needs_layout_passes=False),
    scratch_types=(
        pltpu.VMEM_SHARED((NPAD, D), jnp.float32),
        pltpu.VMEM((3, SUB, D), jnp.float32),
        pltpu.VMEM((2, IGRP, SUB), jnp.int32),
        pltpu.VMEM((HR * D - D,), jnp.float32),
        pltpu.SemaphoreType.DMA((3,)),
        pltpu.SemaphoreType.DMA,
        pltpu.SemaphoreType.DMA,
    ),
)(_sc_body)


def _tc_body(x_ref, s0_ref, s1_ref, ch_ref, b_ref, u_ref,
             W0_ref, W1_ref, W2_ref, b0_ref, b1_ref, b2_ref,
             s0s_ref, s1s_ref, s2s_ref, be0_ref, be1_ref, be2_ref,
             out_ref):
    cnt = jnp.sum(ch_ref[...], axis=0).reshape(BLK, 1)
    v = (s0_ref[...] + s1_ref[...]) / jnp.maximum(cnt, 1.0)
    bidx = b_ref[0, 0, :]
    onehot = (bidx[:, None]
              == lax.broadcasted_iota(jnp.int32, (1, G), 1)).astype(jnp.float32)
    uc = jnp.dot(u_ref[...], W0_ref[2 * D:3 * D, :],
                 preferred_element_type=jnp.float32)
    h = (jnp.dot(x_ref[...], W0_ref[0:D, :], preferred_element_type=jnp.float32)
         + jnp.dot(v, W0_ref[D:2 * D, :], preferred_element_type=jnp.float32)
         + jnp.dot(onehot, uc, preferred_element_type=jnp.float32)
         + b0_ref[...])
    h = jnp.maximum(h, 0.0) * s0s_ref[...] + be0_ref[...]
    h = jnp.dot(h, W1_ref[...], preferred_element_type=jnp.float32) + b1_ref[...]
    h = jnp.maximum(h, 0.0) * s1s_ref[...] + be1_ref[...]
    h = jnp.dot(h, W2_ref[...], preferred_element_type=jnp.float32) + b2_ref[...]
    h = jnp.maximum(h, 0.0) * s2s_ref[...] + be2_ref[...]
    out_ref[...] = h


BLK = 1024
NB = -(-N // BLK)


def _tc_mlp(x, s0, s1, cnth, batch3, u, W0, W1, W2,
            b0, b1, b2, s0s, s1s, s2s, be0, be1, be2):
    row = lambda i: (i, 0)
    full = lambda i: (0, 0)
    vec3 = pl.BlockSpec((1, 1, BLK), lambda i: (i, 0, 0))
    return pl.pallas_call(
        _tc_body,
        grid=(NB,),
        in_specs=[
            pl.BlockSpec((BLK, D), row),
            pl.BlockSpec((BLK, D), row),
            pl.BlockSpec((BLK, D), row),
            pl.BlockSpec((NW, BLK), lambda i: (0, i)),
            vec3,
            pl.BlockSpec((G, D), full),
            pl.BlockSpec((3 * D, D), full),
            pl.BlockSpec((D, D), full),
            pl.BlockSpec((D, D), full),
        ] + [pl.BlockSpec((1, D), full)] * 9,
        out_specs=pl.BlockSpec((BLK, D), row),
        out_shape=jax.ShapeDtypeStruct((N, D), jnp.float32),
    )(x, s0, s1, cnth, batch3, u, W0, W1, W2,
      b0, b1, b2, s0s, s1s, s2s, be0, be1, be2)


def kernel(x, edge_index, edge_attr, u, batch, W0, b0, W1, b1, W2, b2,
           g0, be0, g1, be1, g2, be2):
    src3 = edge_index[0, :].reshape(NW, NSTEP, SUB)
    src3 = jnp.concatenate(
        [src3, jnp.zeros((NW, NGRP * IGRP - NSTEP, SUB), jnp.int32)], axis=1)
    z80 = jnp.zeros((SUB, D), jnp.float32)
    zflat = jnp.zeros((HR * D - D,), jnp.float32)
    sums_p, cnth = _sc_segsum(edge_attr, src3, z80, zflat)

    inv_std = 1.0 / jnp.sqrt(1.0 + EPS)
    r = lambda a: a.reshape(1, D)
    bp = jnp.zeros((NB * BLK,), jnp.int32).at[:N].set(batch)
    return _tc_mlp(
        x, sums_p[0], sums_p[1], cnth,
        bp.reshape(NB, 1, BLK), u, W0, W1, W2,
        r(b0), r(b1), r(b2),
        r(g0 * inv_std), r(g1 * inv_std), r(g2 * inv_std),
        r(be0), r(be1), r(be2))
